# Initial kernel scaffold; baseline (speedup 1.0000x reference)
import functools

import jax
import jax.numpy as jnp
from jax import lax
from jax.experimental import pallas as pl
from jax.experimental.pallas import tpu as pltpu
from jax.experimental.pallas import tpu_sc as plsc

NC = 2
NS = 16
NW = NC * NS
LANES = 16


def _make_seg_sum(n, d, nch, c, with_counts):
  rpt = n // NS

  mesh = plsc.VectorSubcoreMesh(core_axis_name="c", subcore_axis_name="s")

  out_type = [jax.ShapeDtypeStruct((NC, n, d), jnp.float32)]
  if with_counts:
    out_type.append(jax.ShapeDtypeStruct((NC, n, LANES), jnp.float32))

  scratch = [
      pltpu.VMEM((nch, c), jnp.int32),
      pltpu.VMEM((nch, c), jnp.int32),
      pltpu.VMEM((c, d), jnp.float32),
      pltpu.VMEM((c, d), jnp.float32),
      pltpu.VMEM((c, LANES), jnp.float32),
      pltpu.VMEM_SHARED((n, d), jnp.float32),
      pltpu.VMEM_SHARED((n, LANES), jnp.float32),
      pltpu.SemaphoreType.DMA,
      pltpu.SemaphoreType.DMA,
  ]

  @functools.partial(pl.kernel, out_type=out_type, mesh=mesh,
                     scratch_types=scratch)
  def seg_sum(h_hbm, src_hbm, dst_hbm, zrows_hbm, zcnt_hbm, ones_hbm,
              *out_and_scratch):
    if with_counts:
      p_hbm, cnt_hbm = out_and_scratch[0], out_and_scratch[1]
      rest = out_and_scratch[2:]
    else:
      p_hbm = out_and_scratch[0]
      rest = out_and_scratch[1:]
    (src_v, dst_v, rows0, rows1, ones_v, acc_sh, cnt_sh, sem0, sem1) = rest

    cid = lax.axis_index("c")
    sid = lax.axis_index("s")
    g = cid * NS + sid

    pltpu.sync_copy(zrows_hbm, acc_sh.at[pl.ds(sid * rpt, rpt)])
    if with_counts:
      pltpu.sync_copy(zcnt_hbm, cnt_sh.at[pl.ds(sid * rpt, rpt)])
      pltpu.sync_copy(ones_hbm, ones_v)
    pltpu.sync_copy(src_hbm.at[g], src_v)
    pltpu.sync_copy(dst_hbm.at[g], dst_v)
    plsc.subcore_barrier()

    rows = (rows0, rows1)
    sems = (sem0, sem1)

    for b in range(2):
      pltpu.async_copy(h_hbm.at[src_v.at[b]], rows[b], sems[b])

    def step(i2, carry):
      for b in range(2):
        k = i2 * 2 + b
        pltpu.make_async_copy(h_hbm.at[src_v.at[k]], rows[b], sems[b]).wait()
        pltpu.sync_copy(rows[b], acc_sh.at[dst_v.at[k]], add=True)
        if with_counts:
          pltpu.sync_copy(ones_v, cnt_sh.at[dst_v.at[k]], add=True)

        @pl.when(k + 2 < nch)
        def _():
          pltpu.async_copy(h_hbm.at[src_v.at[k + 2]], rows[b], sems[b])
      return carry

    lax.fori_loop(0, nch // 2, step, 0)

    plsc.subcore_barrier()
    pltpu.sync_copy(acc_sh.at[pl.ds(sid * rpt, rpt)],
                    p_hbm.at[cid, pl.ds(sid * rpt, rpt)])
    if with_counts:
      pltpu.sync_copy(cnt_sh.at[pl.ds(sid * rpt, rpt)],
                      cnt_hbm.at[cid, pl.ds(sid * rpt, rpt)])

  return seg_sum


def _relu_body(x_ref, o_ref):
  o_ref[...] = jnp.maximum(x_ref[...], 0.0)


def _relu_tc(x):
  return pl.pallas_call(
      _relu_body,
      out_shape=jax.ShapeDtypeStruct(x.shape, x.dtype),
  )(x)


def _combine_body(p_ref, cnt_ref, h_ref, wn_ref, bn_ref, ws_ref, bs_ref,
                  wu_ref, bu_ref, o_ref, *, hdim, relu):
  s = p_ref[0] + p_ref[1]
  c = cnt_ref[0, :, 0] + cnt_ref[1, :, 0]
  mean = s / jnp.maximum(c, 1.0)[:, None]
  hn = jnp.dot(mean, wn_ref[...], preferred_element_type=jnp.float32)
  hn = hn + bn_ref[0]
  hs = jnp.dot(h_ref[...], ws_ref[...], preferred_element_type=jnp.float32)
  hs = hs + bs_ref[0]
  u = (jnp.dot(hn, wu_ref[:hdim], preferred_element_type=jnp.float32)
       + jnp.dot(hs, wu_ref[hdim:], preferred_element_type=jnp.float32)
       + bu_ref[0])
  o_ref[...] = jnp.maximum(u, 0.0) if relu else u


def _combine_tc(p, cnt, h, wn, bn, ws, bs, wu, bu, relu):
  n, d = h.shape
  hdim = wn.shape[1]
  bn_ = bn.reshape(1, hdim)
  bs_ = bs.reshape(1, hdim)
  bu_ = bu.reshape(1, hdim)
  bnrows = 2500
  grid = (n // bnrows,)
  return pl.pallas_call(
      functools.partial(_combine_body, hdim=hdim, relu=relu),
      grid=grid,
      in_specs=[
          pl.BlockSpec((NC, bnrows, d), lambda i: (0, i, 0)),
          pl.BlockSpec((NC, bnrows, LANES), lambda i: (0, i, 0)),
          pl.BlockSpec((bnrows, d), lambda i: (i, 0)),
          pl.BlockSpec((d, hdim), lambda i: (0, 0)),
          pl.BlockSpec((1, hdim), lambda i: (0, 0)),
          pl.BlockSpec((d, hdim), lambda i: (0, 0)),
          pl.BlockSpec((1, hdim), lambda i: (0, 0)),
          pl.BlockSpec((2 * hdim, hdim), lambda i: (0, 0)),
          pl.BlockSpec((1, hdim), lambda i: (0, 0)),
      ],
      out_specs=pl.BlockSpec((bnrows, hdim), lambda i: (i, 0)),
      out_shape=jax.ShapeDtypeStruct((n, hdim), jnp.float32),
  )(p, cnt, h, wn, bn_, ws, bs_, wu, bu_)


def kernel(x, edge_index, Wn1, bn1, Ws1, bs1, Wu1, bu1,
           Wn2, bn2, Ws2, bs2, Wu2, bu2):
  n, d = x.shape
  e = edge_index.shape[1]
  ew = e // NW
  c = 40
  nch = ew // c
  assert ew * NW == e and nch * c == ew and nch % 2 == 0 and n % NS == 0

  src = edge_index[0].reshape(NW, nch, c)
  dst = edge_index[1].reshape(NW, nch, c)
  zrows = jnp.zeros((n // NS, d), jnp.float32)
  zcnt = jnp.zeros((n // NS, LANES), jnp.float32)
  ones = jnp.ones((c, LANES), jnp.float32)

  seg1 = _make_seg_sum(n, d, nch, c, with_counts=True)
  seg2 = _make_seg_sum(n, d, nch, c, with_counts=False)

  h1 = _relu_tc(x)
  p1, cnt = seg1(h1, src, dst, zrows, zcnt, ones)
  h2 = _combine_tc(p1, cnt, h1, Wn1, bn1, Ws1, bs1, Wu1, bu1, relu=True)
  p2 = seg2(h2, src, dst, zrows, zcnt, ones)
  out = _combine_tc(p2, cnt, h2, Wn2, bn2, Ws2, bs2, Wu2, bu2, relu=False)
  return out


# speedup vs baseline: 6.2049x; 6.2049x over previous
import functools

import jax
import jax.numpy as jnp
from jax import lax
from jax.experimental import pallas as pl
from jax.experimental.pallas import tpu as pltpu
from jax.experimental.pallas import tpu_sc as plsc

NC = 2
NS = 16
LANES = 16


def _make_seg_sum(n, dh, nblk, kb, c, with_counts):
  rpt = 8 * (n // (8 * NS))
  tail = n - rpt * NS

  mesh = plsc.VectorSubcoreMesh(core_axis_name="c", subcore_axis_name="s",
                                num_cores=NC)

  p_type = jax.ShapeDtypeStruct((NC, n, dh), jnp.float32)
  if with_counts:
    out_type = [p_type, jax.ShapeDtypeStruct((n, LANES), jnp.float32)]
  else:
    out_type = p_type

  scratch = [
      pltpu.VMEM((kb, c), jnp.int32),
      pltpu.VMEM((kb, c), jnp.int32),
      pltpu.VMEM((c, dh), jnp.float32),
      pltpu.VMEM((c, dh), jnp.float32),
      pltpu.VMEM((c, LANES), jnp.float32),
      pltpu.VMEM_SHARED((n, dh), jnp.float32),
      pltpu.VMEM_SHARED((n, LANES), jnp.float32),
      pltpu.SemaphoreType.DMA,
      pltpu.SemaphoreType.DMA,
  ]
  if not with_counts:
    del scratch[6]

  @functools.partial(
      pl.kernel, out_type=out_type, mesh=mesh, scratch_types=scratch,
      compiler_params=pltpu.CompilerParams(use_tc_tiling_on_sc=False))
  def seg_sum(h_hbm, src_hbm, dst_hbm, zrows_hbm, zcnt_hbm, ones_hbm,
              *out_and_scratch):
    if with_counts:
      p_hbm, cnt_hbm = out_and_scratch[0], out_and_scratch[1]
      (src_v, dst_v, rows0, rows1, ones_v, acc_sh, cnt_sh,
       sem0, sem1) = out_and_scratch[2:]
    else:
      p_hbm = out_and_scratch[0]
      cnt_hbm = cnt_sh = None
      (src_v, dst_v, rows0, rows1, ones_v, acc_sh,
       sem0, sem1) = out_and_scratch[1:]

    cid = lax.axis_index("c")
    sid = lax.axis_index("s")
    htab = h_hbm.at[cid]

    pltpu.sync_copy(zrows_hbm, acc_sh.at[pl.ds(sid * rpt, rpt)])
    if with_counts:
      pltpu.sync_copy(zcnt_hbm, cnt_sh.at[pl.ds(sid * rpt, rpt)])
      pltpu.sync_copy(ones_hbm, ones_v)
    if tail:
      @pl.when(sid == 0)
      def _():
        pltpu.sync_copy(zrows_hbm.at[pl.ds(0, tail)],
                        acc_sh.at[pl.ds(NS * rpt, tail)])
        if with_counts:
          pltpu.sync_copy(zcnt_hbm.at[pl.ds(0, tail)],
                          cnt_sh.at[pl.ds(NS * rpt, tail)])
    plsc.subcore_barrier()

    rows = (rows0, rows1)
    sems = (sem0, sem1)

    def block(j, carry):
      pltpu.sync_copy(src_hbm.at[sid, j], src_v)
      pltpu.sync_copy(dst_hbm.at[sid, j], dst_v)

      for b in range(2):
        pltpu.async_copy(htab.at[src_v.at[b]], rows[b], sems[b])

      def step(i2, carry2):
        for b in range(2):
          k = i2 * 2 + b
          pltpu.make_async_copy(htab.at[src_v.at[k]], rows[b],
                                sems[b]).wait()
          pltpu.sync_copy(rows[b], acc_sh.at[dst_v.at[k]], add=True)
          if with_counts:
            @pl.when(cid == 0)
            def _():
              pltpu.sync_copy(ones_v, cnt_sh.at[dst_v.at[k]], add=True)

          @pl.when(k + 2 < kb)
          def _():
            pltpu.async_copy(htab.at[src_v.at[k + 2]], rows[b], sems[b])
        return carry2

      lax.fori_loop(0, kb // 2, step, 0)
      return carry

    lax.fori_loop(0, nblk, block, 0)

    plsc.subcore_barrier()
    pltpu.sync_copy(acc_sh.at[pl.ds(sid * rpt, rpt)],
                    p_hbm.at[cid, pl.ds(sid * rpt, rpt)])
    if with_counts:
      @pl.when(cid == 0)
      def _():
        pltpu.sync_copy(cnt_sh.at[pl.ds(sid * rpt, rpt)],
                        cnt_hbm.at[pl.ds(sid * rpt, rpt)])
    if tail:
      @pl.when(sid == 0)
      def _():
        pltpu.sync_copy(acc_sh.at[pl.ds(NS * rpt, tail)],
                        p_hbm.at[cid, pl.ds(NS * rpt, tail)])
        if with_counts:
          @pl.when(cid == 0)
          def _():
            pltpu.sync_copy(cnt_sh.at[pl.ds(NS * rpt, tail)],
                            cnt_hbm.at[pl.ds(NS * rpt, tail)])

    return None

  return seg_sum


def _relu_split_body(x_ref, o_ref, *, dh):
  h = jnp.maximum(x_ref[...], 0.0)
  o_ref[0] = h[:, :dh]
  o_ref[1] = h[:, dh:]


def _relu_split_tc(x, dh):
  n, d = x.shape
  bn = 2000
  return pl.pallas_call(
      functools.partial(_relu_split_body, dh=dh),
      grid=(n // bn,),
      in_specs=[pl.BlockSpec((bn, d), lambda i: (i, 0))],
      out_specs=pl.BlockSpec((NC, bn, dh), lambda i: (0, i, 0)),
      out_shape=jax.ShapeDtypeStruct((NC, n, dh), jnp.float32),
  )(x)


def _combine_body(p_ref, cnt_ref, h_ref, wna_ref, wnb_ref, bn_ref,
                  wsa_ref, wsb_ref, bs_ref, wuna_ref, wunb_ref,
                  wusa_ref, wusb_ref, bua_ref, bub_ref, o_ref, *, relu):
  dot = functools.partial(jnp.dot, preferred_element_type=jnp.float32)
  cden = jnp.maximum(cnt_ref[:, 0], 1.0)[:, None]
  m0 = p_ref[0] / cden
  m1 = p_ref[1] / cden
  hn = dot(m0, wna_ref[...]) + dot(m1, wnb_ref[...]) + bn_ref[0]
  hs = dot(h_ref[0], wsa_ref[...]) + dot(h_ref[1], wsb_ref[...]) + bs_ref[0]
  ua = dot(hn, wuna_ref[...]) + dot(hs, wusa_ref[...]) + bua_ref[0]
  ub = dot(hn, wunb_ref[...]) + dot(hs, wusb_ref[...]) + bub_ref[0]
  if relu:
    o_ref[0] = jnp.maximum(ua, 0.0)
    o_ref[1] = jnp.maximum(ub, 0.0)
  else:
    o_ref[...] = jnp.concatenate([ua, ub], axis=1)


def _combine_tc(p, cnt, h2, wn, bn, ws, bs, wu, bu, relu):
  _, n, dh = h2.shape
  d = 2 * dh
  hdim = wn.shape[1]
  bn_ = bn.reshape(1, hdim)
  bs_ = bs.reshape(1, hdim)
  wna, wnb = wn[:dh], wn[dh:]
  wsa, wsb = ws[:dh], ws[dh:]
  wun, wus = wu[:hdim], wu[hdim:]
  wuna, wunb = wun[:, :dh], wun[:, dh:]
  wusa, wusb = wus[:, :dh], wus[:, dh:]
  bua = bu[:dh].reshape(1, dh)
  bub = bu[dh:].reshape(1, dh)
  bnrows = 2000
  grid = (n // bnrows,)
  if relu:
    out_shape = jax.ShapeDtypeStruct((NC, n, dh), jnp.float32)
    out_spec = pl.BlockSpec((NC, bnrows, dh), lambda i: (0, i, 0))
  else:
    out_shape = jax.ShapeDtypeStruct((n, hdim), jnp.float32)
    out_spec = pl.BlockSpec((bnrows, hdim), lambda i: (i, 0))
  full = lambda a: pl.BlockSpec(a.shape, lambda i: (0,) * a.ndim)
  return pl.pallas_call(
      functools.partial(_combine_body, relu=relu),
      grid=grid,
      in_specs=[
          pl.BlockSpec((NC, bnrows, dh), lambda i: (0, i, 0)),
          pl.BlockSpec((bnrows, LANES), lambda i: (i, 0)),
          pl.BlockSpec((NC, bnrows, dh), lambda i: (0, i, 0)),
          full(wna), full(wnb), full(bn_),
          full(wsa), full(wsb), full(bs_),
          full(wuna), full(wunb), full(wusa), full(wusb),
          full(bua), full(bub),
      ],
      out_specs=out_spec,
      out_shape=out_shape,
  )(p, cnt, h2, wna, wnb, bn_, wsa, wsb, bs_, wuna, wunb, wusa, wusb,
    bua, bub)


def kernel(x, edge_index, Wn1, bn1, Ws1, bs1, Wu1, bu1,
           Wn2, bn2, Ws2, bs2, Wu2, bu2):
  n, d = x.shape
  e = edge_index.shape[1]
  dh = d // NC
  ept = e // NS
  c = 40
  kb = 50
  nblk = ept // (kb * c)
  assert dh * NC == d and ept * NS == e and nblk * kb * c == ept
  assert n % NS == 0

  src = edge_index[0].reshape(NS, nblk, kb, c)
  dst = edge_index[1].reshape(NS, nblk, kb, c)
  rpt = 8 * (n // (8 * NS))
  assert 0 <= n - rpt * NS <= rpt
  zrows = jnp.zeros((rpt, dh), jnp.float32)
  zcnt = jnp.zeros((rpt, LANES), jnp.float32)
  ones = jnp.ones((c, LANES), jnp.float32)

  seg1 = _make_seg_sum(n, dh, nblk, kb, c, with_counts=True)
  seg2 = _make_seg_sum(n, dh, nblk, kb, c, with_counts=False)

  h1 = _relu_split_tc(x, dh)
  p1, cnt = seg1(h1, src, dst, zrows, zcnt, ones)
  h2 = _combine_tc(p1, cnt, h1, Wn1, bn1, Ws1, bs1, Wu1, bu1, relu=True)
  p2 = seg2(h2, src, dst, zrows, zcnt, ones)
  out = _combine_tc(p2, cnt, h2, Wn2, bn2, Ws2, bs2, Wu2, bu2, relu=False)
  return out


# speedup vs baseline: 9.8604x; 1.5891x over previous
import functools

import jax
import jax.numpy as jnp
from jax import lax
from jax.experimental import pallas as pl
from jax.experimental.pallas import tpu as pltpu
from jax.experimental.pallas import tpu_sc as plsc

NC = 2
NS = 16
LANES = 16


def _make_seg_sum(n, dh, nblk, kb, c, with_counts):
  rpt = 8 * (n // (8 * NS))
  tail = n - rpt * NS

  mesh = plsc.VectorSubcoreMesh(core_axis_name="c", subcore_axis_name="s",
                                num_cores=NC)

  p_type = jax.ShapeDtypeStruct((NC, n, dh), jnp.float32)
  if with_counts:
    out_type = [p_type, jax.ShapeDtypeStruct((n, LANES), jnp.float32)]
  else:
    out_type = p_type

  nbuf = 4
  scratch = (
      [pltpu.VMEM((kb, c), jnp.int32)] * 2
      + [pltpu.VMEM((c, dh), jnp.float32)] * nbuf
      + [pltpu.VMEM((c, LANES), jnp.float32)]
      + [pltpu.VMEM_SHARED((n, dh), jnp.float32)]
      + ([pltpu.VMEM_SHARED((n, LANES), jnp.float32)]
         if with_counts else [])
      + [pltpu.SemaphoreType.DMA] * (2 * nbuf)
  )

  @functools.partial(
      pl.kernel, out_type=out_type, mesh=mesh, scratch_types=scratch,
      compiler_params=pltpu.CompilerParams(use_tc_tiling_on_sc=False))
  def seg_sum(h_hbm, src_hbm, dst_hbm, zrows_hbm, zcnt_hbm, ones_hbm,
              *out_and_scratch):
    if with_counts:
      p_hbm, cnt_hbm = out_and_scratch[0], out_and_scratch[1]
      (src_v, dst_v, *rest) = out_and_scratch[2:]
      rows = rest[:nbuf]
      ones_v, acc_sh, cnt_sh = rest[nbuf:nbuf + 3]
      sems = rest[nbuf + 3:]
    else:
      p_hbm = out_and_scratch[0]
      cnt_hbm = cnt_sh = None
      (src_v, dst_v, *rest) = out_and_scratch[1:]
      rows = rest[:nbuf]
      ones_v, acc_sh = rest[nbuf:nbuf + 2]
      sems = rest[nbuf + 2:]
    gsem = sems[:nbuf]
    ssem = sems[nbuf:]

    cid = lax.axis_index("c")
    sid = lax.axis_index("s")
    htab = h_hbm.at[cid]

    pltpu.sync_copy(zrows_hbm, acc_sh.at[pl.ds(sid * rpt, rpt)])
    if with_counts:
      pltpu.sync_copy(zcnt_hbm, cnt_sh.at[pl.ds(sid * rpt, rpt)])
      pltpu.sync_copy(ones_hbm, ones_v)
    if tail:
      @pl.when(sid == 0)
      def _():
        pltpu.sync_copy(zrows_hbm.at[pl.ds(0, tail)],
                        acc_sh.at[pl.ds(NS * rpt, tail)])
        if with_counts:
          pltpu.sync_copy(zcnt_hbm.at[pl.ds(0, tail)],
                          cnt_sh.at[pl.ds(NS * rpt, tail)])
    plsc.subcore_barrier()

    def block(j, carry):
      pltpu.sync_copy(src_hbm.at[sid, j], src_v)
      pltpu.sync_copy(dst_hbm.at[sid, j], dst_v)

      for b in range(nbuf):
        pltpu.async_copy(htab.at[src_v.at[b]], rows[b], gsem[b])

      def step(i2, carry2):
        for b in range(nbuf):
          k = i2 * nbuf + b
          pltpu.make_async_copy(htab.at[src_v.at[k]], rows[b],
                                gsem[b]).wait()
          pltpu.async_copy(rows[b], acc_sh.at[dst_v.at[k]], ssem[b],
                           add=True)
          if with_counts:
            @pl.when(cid == 0)
            def _():
              pltpu.sync_copy(ones_v, cnt_sh.at[dst_v.at[k]], add=True)
          pltpu.make_async_copy(rows[b], acc_sh.at[dst_v.at[k]],
                                ssem[b]).wait()

          @pl.when(k + nbuf < kb)
          def _():
            pltpu.async_copy(htab.at[src_v.at[k + nbuf]], rows[b], gsem[b])
        return carry2

      lax.fori_loop(0, kb // nbuf, step, 0)
      return carry

    lax.fori_loop(0, nblk, block, 0)

    plsc.subcore_barrier()
    pltpu.sync_copy(acc_sh.at[pl.ds(sid * rpt, rpt)],
                    p_hbm.at[cid, pl.ds(sid * rpt, rpt)])
    if with_counts:
      @pl.when(cid == 0)
      def _():
        pltpu.sync_copy(cnt_sh.at[pl.ds(sid * rpt, rpt)],
                        cnt_hbm.at[pl.ds(sid * rpt, rpt)])
    if tail:
      @pl.when(sid == 0)
      def _():
        pltpu.sync_copy(acc_sh.at[pl.ds(NS * rpt, tail)],
                        p_hbm.at[cid, pl.ds(NS * rpt, tail)])
        if with_counts:
          @pl.when(cid == 0)
          def _():
            pltpu.sync_copy(cnt_sh.at[pl.ds(NS * rpt, tail)],
                            cnt_hbm.at[pl.ds(NS * rpt, tail)])

    return None

  return seg_sum


def _relu_split_body(x_ref, o_ref, *, dh):
  h = jnp.maximum(x_ref[...], 0.0)
  o_ref[0] = h[:, :dh]
  o_ref[1] = h[:, dh:]


def _relu_split_tc(x, dh):
  n, d = x.shape
  bn = 2000
  return pl.pallas_call(
      functools.partial(_relu_split_body, dh=dh),
      grid=(n // bn,),
      in_specs=[pl.BlockSpec((bn, d), lambda i: (i, 0))],
      out_specs=pl.BlockSpec((NC, bn, dh), lambda i: (0, i, 0)),
      out_shape=jax.ShapeDtypeStruct((NC, n, dh), jnp.float32),
  )(x)


def _combine_body(p_ref, cnt_ref, h_ref, wna_ref, wnb_ref, bn_ref,
                  wsa_ref, wsb_ref, bs_ref, wuna_ref, wunb_ref,
                  wusa_ref, wusb_ref, bua_ref, bub_ref, o_ref, *, relu):
  dot = functools.partial(jnp.dot, preferred_element_type=jnp.float32)
  cden = jnp.maximum(cnt_ref[:, 0], 1.0)[:, None]
  m0 = p_ref[0] / cden
  m1 = p_ref[1] / cden
  hn = dot(m0, wna_ref[...]) + dot(m1, wnb_ref[...]) + bn_ref[0]
  hs = dot(h_ref[0], wsa_ref[...]) + dot(h_ref[1], wsb_ref[...]) + bs_ref[0]
  ua = dot(hn, wuna_ref[...]) + dot(hs, wusa_ref[...]) + bua_ref[0]
  ub = dot(hn, wunb_ref[...]) + dot(hs, wusb_ref[...]) + bub_ref[0]
  if relu:
    o_ref[0] = jnp.maximum(ua, 0.0)
    o_ref[1] = jnp.maximum(ub, 0.0)
  else:
    o_ref[...] = jnp.concatenate([ua, ub], axis=1)


def _combine_tc(p, cnt, h2, wn, bn, ws, bs, wu, bu, relu):
  _, n, dh = h2.shape
  d = 2 * dh
  hdim = wn.shape[1]
  bn_ = bn.reshape(1, hdim)
  bs_ = bs.reshape(1, hdim)
  wna, wnb = wn[:dh], wn[dh:]
  wsa, wsb = ws[:dh], ws[dh:]
  wun, wus = wu[:hdim], wu[hdim:]
  wuna, wunb = wun[:, :dh], wun[:, dh:]
  wusa, wusb = wus[:, :dh], wus[:, dh:]
  bua = bu[:dh].reshape(1, dh)
  bub = bu[dh:].reshape(1, dh)
  bnrows = 2000
  grid = (n // bnrows,)
  if relu:
    out_shape = jax.ShapeDtypeStruct((NC, n, dh), jnp.float32)
    out_spec = pl.BlockSpec((NC, bnrows, dh), lambda i: (0, i, 0))
  else:
    out_shape = jax.ShapeDtypeStruct((n, hdim), jnp.float32)
    out_spec = pl.BlockSpec((bnrows, hdim), lambda i: (i, 0))
  full = lambda a: pl.BlockSpec(a.shape, lambda i: (0,) * a.ndim)
  return pl.pallas_call(
      functools.partial(_combine_body, relu=relu),
      grid=grid,
      in_specs=[
          pl.BlockSpec((NC, bnrows, dh), lambda i: (0, i, 0)),
          pl.BlockSpec((bnrows, LANES), lambda i: (i, 0)),
          pl.BlockSpec((NC, bnrows, dh), lambda i: (0, i, 0)),
          full(wna), full(wnb), full(bn_),
          full(wsa), full(wsb), full(bs_),
          full(wuna), full(wunb), full(wusa), full(wusb),
          full(bua), full(bub),
      ],
      out_specs=out_spec,
      out_shape=out_shape,
  )(p, cnt, h2, wna, wnb, bn_, wsa, wsb, bs_, wuna, wunb, wusa, wusb,
    bua, bub)


def kernel(x, edge_index, Wn1, bn1, Ws1, bs1, Wu1, bu1,
           Wn2, bn2, Ws2, bs2, Wu2, bu2):
  n, d = x.shape
  e = edge_index.shape[1]
  dh = d // NC
  ept = e // NS
  c = 40
  kb = 100
  nblk = ept // (kb * c)
  assert dh * NC == d and ept * NS == e and nblk * kb * c == ept
  assert n % NS == 0

  src = edge_index[0].reshape(NS, nblk, kb, c)
  dst = edge_index[1].reshape(NS, nblk, kb, c)
  rpt = 8 * (n // (8 * NS))
  assert 0 <= n - rpt * NS <= rpt
  zrows = jnp.zeros((rpt, dh), jnp.float32)
  zcnt = jnp.zeros((rpt, LANES), jnp.float32)
  ones = jnp.ones((c, LANES), jnp.float32)

  seg1 = _make_seg_sum(n, dh, nblk, kb, c, with_counts=True)
  seg2 = _make_seg_sum(n, dh, nblk, kb, c, with_counts=False)

  h1 = _relu_split_tc(x, dh)
  p1, cnt = seg1(h1, src, dst, zrows, zcnt, ones)
  h2 = _combine_tc(p1, cnt, h1, Wn1, bn1, Ws1, bs1, Wu1, bu1, relu=True)
  p2 = seg2(h2, src, dst, zrows, zcnt, ones)
  out = _combine_tc(p2, cnt, h2, Wn2, bn2, Ws2, bs2, Wu2, bu2, relu=False)
  return out
